# Initial kernel scaffold; baseline (speedup 1.0000x reference)
import jax
import jax.numpy as jnp
from jax.experimental import pallas as pl


def kernel(boxes, classification):
    raise NotImplementedError("write your pallas kernel here")


# speedup vs baseline: 12.7873x; 12.7873x over previous
import functools

import jax
import jax.numpy as jnp
from jax import lax
from jax.experimental import pallas as pl
from jax.experimental.pallas import tpu as pltpu

_NMS_THRESHOLD = 0.5
_SCORE_THRESHOLD = 0.05
_MAX_BOXES = 300

_N_PAD = 5120


def _nms_body(bx_ref, cls_ref, out_ref, s_ref):
    neg = jnp.float32(-jnp.inf)
    cls = cls_ref[...]
    s_ref[...] = jnp.where(cls > _SCORE_THRESHOLD, cls, neg)
    out_ref[...] = jnp.zeros_like(out_ref)

    y1 = bx_ref[0:1, :]
    x1 = bx_ref[1:2, :]
    y2 = bx_ref[2:3, :]
    x2 = bx_ref[3:4, :]
    area = (y2 - y1) * (x2 - x1)
    C = out_ref.shape[0]
    col = lax.broadcasted_iota(jnp.int32, (C, _N_PAD), 1)

    def step(_, carry):
        s = s_ref[...]
        m = jnp.max(s, axis=1, keepdims=True)
        valid = m != neg
        eq = jnp.logical_and(s == m, valid)
        idx = jnp.min(jnp.where(eq, col, _N_PAD), axis=1, keepdims=True)
        onehot = col == idx
        zero = jnp.float32(0.0)
        by1 = jnp.sum(jnp.where(onehot, y1, zero), axis=1, keepdims=True)
        bx1 = jnp.sum(jnp.where(onehot, x1, zero), axis=1, keepdims=True)
        by2 = jnp.sum(jnp.where(onehot, y2, zero), axis=1, keepdims=True)
        bx2 = jnp.sum(jnp.where(onehot, x2, zero), axis=1, keepdims=True)
        yy1 = jnp.maximum(by1, y1)
        xx1 = jnp.maximum(bx1, x1)
        yy2 = jnp.minimum(by2, y2)
        xx2 = jnp.minimum(bx2, x2)
        inter = jnp.maximum(yy2 - yy1, zero) * jnp.maximum(xx2 - xx1, zero)
        area_a = (by2 - by1) * (bx2 - bx1)
        iou = inter / (area_a + area - inter + 1e-8)
        supp = jnp.logical_or(iou > _NMS_THRESHOLD, onehot)
        s_ref[...] = jnp.where(jnp.logical_and(valid, supp), neg, s)
        out_ref[...] = jnp.where(onehot, m, out_ref[...])
        return carry

    lax.fori_loop(0, _MAX_BOXES, step, 0)


@jax.jit
def kernel(boxes, classification):
    n = boxes.shape[1]
    c = classification.shape[2]
    bx = jnp.transpose(boxes[0])
    bx = jnp.pad(bx, ((0, 4), (0, _N_PAD - n)))
    cls = jnp.transpose(classification[0])
    cls = jnp.pad(cls, ((0, 0), (0, _N_PAD - n)))

    out = pl.pallas_call(
        _nms_body,
        out_shape=jax.ShapeDtypeStruct((c, _N_PAD), jnp.float32),
        scratch_shapes=[pltpu.VMEM((c, _N_PAD), jnp.float32)],
    )(bx, cls)
    return jnp.transpose(out[:, :n])[None]


# speedup vs baseline: 15.7637x; 1.2328x over previous
import functools

import jax
import jax.numpy as jnp
from jax import lax
from jax.experimental import pallas as pl
from jax.experimental.pallas import tpu as pltpu

_NMS_THRESHOLD = 0.5
_SCORE_THRESHOLD = 0.05
_MAX_BOXES = 300

_N_PAD = 5120
_NW = 640
_NK = _N_PAD // _NW
_RB = 128


def _adj_body(bx_ref, bxt_ref, mf_ref):
    y1 = bx_ref[0:1, :]
    x1 = bx_ref[1:2, :]
    y2 = bx_ref[2:3, :]
    x2 = bx_ref[3:4, :]
    area = (y2 - y1) * (x2 - x1)
    ry1 = bxt_ref[:, 0:1]
    rx1 = bxt_ref[:, 1:2]
    ry2 = bxt_ref[:, 2:3]
    rx2 = bxt_ref[:, 3:4]
    zero = jnp.float32(0.0)
    yy1 = jnp.maximum(ry1, y1)
    xx1 = jnp.maximum(rx1, x1)
    yy2 = jnp.minimum(ry2, y2)
    xx2 = jnp.minimum(rx2, x2)
    inter = jnp.maximum(yy2 - yy1, zero) * jnp.maximum(xx2 - xx1, zero)
    area_a = (ry2 - ry1) * (rx2 - rx1)
    iou = inter / (area_a + area - inter + 1e-8)
    supp = (iou > _NMS_THRESHOLD).astype(jnp.int32)
    words = jnp.zeros((_RB, _NW), jnp.int32)
    for k in range(_NK):
        words = words | (supp[:, k * _NW:(k + 1) * _NW] << k)
    mf_ref[...] = words.astype(jnp.float32)


def _greedy_body(cls_ref, mf_ref, out_ref, s_ref):
    neg = jnp.float32(-jnp.inf)
    cls = cls_ref[...]
    s_ref[...] = jnp.where(cls > _SCORE_THRESHOLD, cls, neg)
    out_ref[...] = jnp.zeros_like(out_ref)
    C = out_ref.shape[0]
    col = lax.broadcasted_iota(jnp.int32, (C, _N_PAD), 1)

    def step(_, carry):
        s = s_ref[...]
        m = jnp.max(s, axis=1, keepdims=True)
        valid = m != neg
        eq = jnp.logical_and(s == m, valid)
        idx = jnp.min(jnp.where(eq, col, _N_PAD), axis=1, keepdims=True)
        onehot = col == idx
        oh = onehot.astype(jnp.float32)
        rows = jax.lax.dot_general(
            oh, mf_ref[...], (((1,), (0,)), ((), ())),
            preferred_element_type=jnp.float32)
        w = rows.astype(jnp.int32)
        planes = [((w >> k) & 1) for k in range(_NK)]
        bits = jnp.concatenate(planes, axis=1) == 1
        supp = jnp.logical_or(bits, onehot)
        s_ref[...] = jnp.where(jnp.logical_and(valid, supp), neg, s)
        out_ref[...] = jnp.where(onehot, m, out_ref[...])
        return carry

    lax.fori_loop(0, _MAX_BOXES, step, 0)


@jax.jit
def kernel(boxes, classification):
    n = boxes.shape[1]
    c = classification.shape[2]
    bx = jnp.transpose(boxes[0])
    bx = jnp.pad(bx, ((0, 4), (0, _N_PAD - n)))
    bxt = jnp.pad(boxes[0], ((0, _N_PAD - n), (0, 4)))
    cls = jnp.transpose(classification[0])
    cls = jnp.pad(cls, ((0, 0), (0, _N_PAD - n)))

    mf = pl.pallas_call(
        _adj_body,
        grid=(_N_PAD // _RB,),
        in_specs=[
            pl.BlockSpec((8, _N_PAD), lambda i: (0, 0)),
            pl.BlockSpec((_RB, 8), lambda i: (i, 0)),
        ],
        out_specs=pl.BlockSpec((_RB, _NW), lambda i: (i, 0)),
        out_shape=jax.ShapeDtypeStruct((_N_PAD, _NW), jnp.float32),
    )(bx, bxt)

    out = pl.pallas_call(
        _greedy_body,
        out_shape=jax.ShapeDtypeStruct((c, _N_PAD), jnp.float32),
        scratch_shapes=[pltpu.VMEM((c, _N_PAD), jnp.float32)],
    )(cls, mf)
    return jnp.transpose(out[:, :n])[None]
